# baseline (speedup 1.0000x reference)
import functools

import jax
import jax.numpy as jnp
from jax import lax
from jax.experimental import pallas as pl
from jax.experimental.pallas import tpu as pltpu
from jax.experimental.pallas import tpu_sc as plsc

_NC = 2
_NS = 16
_NW = _NC * _NS
_LANES = 16
_BLK = 2048


def _scores_tc(emb, wb, inv_s):
    V, D = emb.shape
    grid = (V + _BLK - 1) // _BLK

    def body(e_ref, w_ref, o_ref):
        i = pl.program_id(0)
        e = e_ref[...]
        row = jax.lax.broadcasted_iota(jnp.int32, (_BLK, 1), 0) + i * _BLK
        e = jnp.where(row < V, e, 0.0)
        sb = jnp.dot(e, w_ref[...],
                     preferred_element_type=jnp.float32,
                     precision=lax.Precision.HIGHEST)
        lane = jax.lax.broadcasted_iota(jnp.int32, (_BLK, 128), 1)
        rmod = jax.lax.broadcasted_iota(jnp.int32, (_BLK, 128), 0) % 128
        masked = jnp.where(lane == rmod, sb, 0.0)
        g_of_r = jax.lax.broadcasted_iota(jnp.int32, (_BLK // 128, _BLK), 1)
        g_id = jax.lax.broadcasted_iota(jnp.int32, (_BLK // 128, _BLK), 0)
        a = jnp.where(g_of_r // 128 == g_id, 1.0, 0.0)
        outc = jnp.dot(a, masked,
                       preferred_element_type=jnp.float32,
                       precision=lax.Precision.HIGHEST)
        o_ref[...] = outc.reshape(_BLK) * inv_s

    return pl.pallas_call(
        body,
        grid=(grid,),
        in_specs=[
            pl.BlockSpec((_BLK, D), lambda i: (i, 0)),
            pl.BlockSpec((D, 128), lambda i: (0, 0)),
        ],
        out_specs=pl.BlockSpec((_BLK,), lambda i: (i,)),
        out_shape=jax.ShapeDtypeStruct((V,), jnp.float32),
    )(emb, wb)


def _transpose_ids_tc(ids3f, S, rows_per_tile):
    def body(x_ref, o_ref):
        eye = jnp.where(
            jax.lax.broadcasted_iota(jnp.int32, (rows_per_tile,) * 2, 0)
            == jax.lax.broadcasted_iota(jnp.int32, (rows_per_tile,) * 2, 1),
            1.0, 0.0)
        xt = lax.dot_general(x_ref[0], eye, (((0,), (0,)), ((), ())),
                             precision=lax.Precision.HIGHEST,
                             preferred_element_type=jnp.float32)
        o_ref[0] = xt.astype(jnp.int32)

    return pl.pallas_call(
        body,
        grid=(_NW,),
        in_specs=[pl.BlockSpec((1, rows_per_tile, S), lambda i: (i, 0, 0))],
        out_specs=pl.BlockSpec((1, S, rows_per_tile), lambda i: (i, 0, 0)),
        out_shape=jax.ShapeDtypeStruct((_NW, S, rows_per_tile), jnp.int32),
    )(ids3f)


def _make_sc_pool(B, S, V):
    rows_per_tile = B // _NW
    groups = rows_per_tile // _LANES
    mesh = plsc.VectorSubcoreMesh(core_axis_name="c", subcore_axis_name="s")

    @functools.partial(
        pl.kernel,
        mesh=mesh,
        out_type=jax.ShapeDtypeStruct((B,), jnp.float32),
        scratch_types=[
            pltpu.VMEM((S, rows_per_tile), jnp.int32),
            pltpu.VMEM((S, rows_per_tile), jnp.float32),
            pltpu.VMEM((rows_per_tile,), jnp.float32),
            pltpu.VMEM((_LANES,), jnp.float32),
            pltpu.SemaphoreType.DMA,
        ],
    )
    def sc_pool(scores_hbm, ids_hbm, b_hbm, out_hbm,
                idx_v, vals_v, outb_v, b_v, sem):
        wid = lax.axis_index("s") * _NC + lax.axis_index("c")
        pltpu.sync_copy(ids_hbm.at[wid], idx_v)
        pltpu.sync_copy(b_hbm, b_v)

        def fire(j, carry):
            pltpu.async_copy(scores_hbm.at[idx_v.at[j]], vals_v.at[j], sem)
            return carry

        lax.fori_loop(0, S, fire, 0)

        def drain(j, carry):
            pltpu.make_async_copy(scores_hbm.at[idx_v.at[j]],
                                  vals_v.at[j], sem).wait()
            return carry

        lax.fori_loop(0, S, drain, 0)

        bias = b_v[...]
        for g in range(groups):
            def red(s, acc, _g=g):
                return acc + vals_v[s, pl.ds(_g * _LANES, _LANES)]

            acc = lax.fori_loop(0, S, red, jnp.zeros((_LANES,), jnp.float32))
            outb_v[pl.ds(g * _LANES, _LANES)] = acc + bias

        pltpu.sync_copy(outb_v, out_hbm.at[pl.ds(wid * rows_per_tile,
                                                 rows_per_tile)])

    return sc_pool


def kernel(input_ids, embedding, head_w, head_b):
    B, S = input_ids.shape
    V, D = embedding.shape
    assert D == 32 and B % (_NW * _LANES) == 0

    wb = jnp.broadcast_to(head_w, (D, 128))
    scores = _scores_tc(embedding, wb, 1.0 / S)

    rows_per_tile = B // _NW
    ids3f = input_ids.reshape(_NW, rows_per_tile, S).astype(jnp.float32)
    ids_t = _transpose_ids_tc(ids3f, S, rows_per_tile)
    b16 = jnp.broadcast_to(head_b.astype(jnp.float32), (_LANES,))

    return _make_sc_pool(B, S, V)(scores, ids_t, b16)


# speedup vs baseline: 1.5853x; 1.5853x over previous
import functools

import jax
import jax.numpy as jnp
from jax import lax
from jax.experimental import pallas as pl
from jax.experimental.pallas import tpu as pltpu
from jax.experimental.pallas import tpu_sc as plsc

_NC = 2
_NS = 16
_NW = _NC * _NS
_LANES = 16
_BLK = 2048


def _scores_tc(emb, wrow, inv_s):
    V, D = emb.shape
    grid = (V + _BLK - 1) // _BLK

    def body(e_ref, w_ref, o_ref):
        s = lax.dot_general(w_ref[...], e_ref[...],
                            (((1,), (1,)), ((), ())),
                            preferred_element_type=jnp.float32,
                            precision=lax.Precision.HIGHEST)
        o_ref[...] = s.reshape(_BLK) * inv_s

    return pl.pallas_call(
        body,
        grid=(grid,),
        in_specs=[
            pl.BlockSpec((_BLK, D), lambda i: (i, 0)),
            pl.BlockSpec((1, D), lambda i: (0, 0)),
        ],
        out_specs=pl.BlockSpec((_BLK,), lambda i: (i,)),
        out_shape=jax.ShapeDtypeStruct((V,), jnp.float32),
    )(emb, wrow)


def _make_sc_pool(B, S, V):
    rows_per_tile = B // _NW
    groups = rows_per_tile // _LANES
    mesh = plsc.VectorSubcoreMesh(core_axis_name="c", subcore_axis_name="s")

    @functools.partial(
        pl.kernel,
        mesh=mesh,
        out_type=jax.ShapeDtypeStruct((B,), jnp.float32),
        scratch_types=[
            pltpu.VMEM((S, rows_per_tile), jnp.int32),
            pltpu.VMEM((S, rows_per_tile), jnp.float32),
            pltpu.VMEM((rows_per_tile,), jnp.float32),
            pltpu.VMEM((_LANES,), jnp.float32),
            pltpu.SemaphoreType.DMA,
        ],
    )
    def sc_pool(scores_hbm, ids_hbm, b_hbm, out_hbm,
                idx_v, vals_v, outb_v, b_v, sem):
        wid = lax.axis_index("s") * _NC + lax.axis_index("c")
        pltpu.sync_copy(ids_hbm.at[wid], idx_v)
        pltpu.sync_copy(b_hbm, b_v)

        def fire(j, carry):
            pltpu.async_copy(scores_hbm.at[idx_v.at[j]], vals_v.at[j], sem)
            return carry

        lax.fori_loop(0, S, fire, 0)

        def drain(j, carry):
            pltpu.make_async_copy(scores_hbm.at[idx_v.at[j]],
                                  vals_v.at[j], sem).wait()
            return carry

        lax.fori_loop(0, S, drain, 0)

        bias = b_v[...]
        for g in range(groups):
            def red(s, acc, _g=g):
                return acc + vals_v[s, pl.ds(_g * _LANES, _LANES)]

            acc = lax.fori_loop(0, S, red, jnp.zeros((_LANES,), jnp.float32))
            outb_v[pl.ds(g * _LANES, _LANES)] = acc + bias

        pltpu.sync_copy(outb_v, out_hbm.at[pl.ds(wid * rows_per_tile,
                                                 rows_per_tile)])

    return sc_pool


def kernel(input_ids, embedding, head_w, head_b):
    B, S = input_ids.shape
    V, D = embedding.shape
    assert D == 32 and B % (_NW * _LANES) == 0

    scores = _scores_tc(embedding, head_w.reshape(1, D), 1.0 / S)

    rows_per_tile = B // _NW
    ids_t = jnp.swapaxes(
        input_ids.reshape(_NW, rows_per_tile, S), 1, 2)
    b16 = jnp.broadcast_to(head_b.astype(jnp.float32), (_LANES,))

    return _make_sc_pool(B, S, V)(scores, ids_t, b16)


# speedup vs baseline: 1.8409x; 1.1612x over previous
import functools

import jax
import jax.numpy as jnp
from jax import lax
from jax.experimental import pallas as pl
from jax.experimental.pallas import tpu as pltpu
from jax.experimental.pallas import tpu_sc as plsc

_NC = 2
_NS = 16
_NW = _NC * _NS
_LANES = 16
_BLK = 32768


def _scores_tc(emb, wrow, inv_s):
    V, D = emb.shape
    grid = (V + _BLK - 1) // _BLK

    def body(e_ref, w_ref, o_ref):
        s = lax.dot_general(w_ref[...], e_ref[...],
                            (((1,), (1,)), ((), ())),
                            preferred_element_type=jnp.float32,
                            precision=lax.Precision.HIGHEST)
        o_ref[...] = s.reshape(_BLK) * inv_s

    return pl.pallas_call(
        body,
        grid=(grid,),
        in_specs=[
            pl.BlockSpec((_BLK, D), lambda i: (i, 0)),
            pl.BlockSpec((1, D), lambda i: (0, 0)),
        ],
        out_specs=pl.BlockSpec((_BLK,), lambda i: (i,)),
        out_shape=jax.ShapeDtypeStruct((V,), jnp.float32),
    )(emb, wrow)


def _make_sc_pool(B, S, V):
    rows_per_tile = B // _NW
    groups = rows_per_tile // _LANES
    mesh = plsc.VectorSubcoreMesh(core_axis_name="c", subcore_axis_name="s")

    @functools.partial(
        pl.kernel,
        mesh=mesh,
        out_type=jax.ShapeDtypeStruct((B,), jnp.float32),
        scratch_types=[
            pltpu.VMEM((S, rows_per_tile), jnp.int32),
            pltpu.VMEM((S, rows_per_tile), jnp.float32),
            pltpu.VMEM((rows_per_tile,), jnp.float32),
            pltpu.VMEM((_LANES,), jnp.float32),
            pltpu.SemaphoreType.DMA,
        ],
    )
    def sc_pool(scores_hbm, ids_hbm, b_hbm, out_hbm,
                idx_v, vals_v, outb_v, b_v, sem):
        wid = lax.axis_index("s") * _NC + lax.axis_index("c")
        pltpu.sync_copy(ids_hbm.at[wid], idx_v)
        pltpu.sync_copy(b_hbm, b_v)

        def fire(j, carry):
            pltpu.async_copy(scores_hbm.at[idx_v.at[j]], vals_v.at[j], sem)
            return carry

        lax.fori_loop(0, S, fire, 0)

        def drain(j, carry):
            pltpu.make_async_copy(scores_hbm.at[idx_v.at[j]],
                                  vals_v.at[j], sem).wait()
            return carry

        lax.fori_loop(0, S, drain, 0)

        bias = b_v[...]
        for g in range(groups):
            def red(s, acc, _g=g):
                return acc + vals_v[s, pl.ds(_g * _LANES, _LANES)]

            acc = lax.fori_loop(0, S, red, jnp.zeros((_LANES,), jnp.float32))
            outb_v[pl.ds(g * _LANES, _LANES)] = acc + bias

        pltpu.sync_copy(outb_v, out_hbm.at[pl.ds(wid * rows_per_tile,
                                                 rows_per_tile)])

    return sc_pool


def kernel(input_ids, embedding, head_w, head_b):
    B, S = input_ids.shape
    V, D = embedding.shape
    assert D == 32 and B % (_NW * _LANES) == 0

    scores = _scores_tc(embedding, head_w.reshape(1, D), 1.0 / S)

    rows_per_tile = B // _NW
    ids_t = jnp.swapaxes(
        input_ids.reshape(_NW, rows_per_tile, S), 1, 2)
    b16 = jnp.broadcast_to(head_b.astype(jnp.float32), (_LANES,))

    return _make_sc_pool(B, S, V)(scores, ids_t, b16)


# speedup vs baseline: 2.0664x; 1.1225x over previous
import functools

import jax
import jax.numpy as jnp
from jax import lax
from jax.experimental import pallas as pl
from jax.experimental.pallas import tpu as pltpu
from jax.experimental.pallas import tpu_sc as plsc

_NC = 2
_NS = 16
_NW = _NC * _NS
_LANES = 16
_FOLD = 8
_BLKR = 8192


def _scores_tc(emb8, wrep):
    N, K = emb8.shape
    grid = (N + _BLKR - 1) // _BLKR

    def body(w_ref, e_ref, o_ref):
        o_ref[...] = lax.dot_general(
            w_ref[...], e_ref[...],
            (((1,), (1,)), ((), ())),
            preferred_element_type=jnp.float32,
            precision=lax.Precision.HIGHEST)

    return pl.pallas_call(
        body,
        grid=(grid,),
        in_specs=[
            pl.BlockSpec((_FOLD, K), lambda i: (0, 0)),
            pl.BlockSpec((_BLKR, K), lambda i: (i, 0)),
        ],
        out_specs=pl.BlockSpec((_FOLD, _BLKR), lambda i: (0, i)),
        out_shape=jax.ShapeDtypeStruct((_FOLD, N), jnp.float32),
    )(wrep, emb8)


def _make_sc_pool(B, S):
    rows_per_tile = B // _NW
    groups = rows_per_tile // _LANES
    mesh = plsc.VectorSubcoreMesh(core_axis_name="c", subcore_axis_name="s")

    @functools.partial(
        pl.kernel,
        mesh=mesh,
        out_type=jax.ShapeDtypeStruct((B,), jnp.float32),
        scratch_types=[
            pltpu.VMEM((S, rows_per_tile), jnp.int32),
            pltpu.VMEM((S, rows_per_tile), jnp.float32),
            pltpu.VMEM((rows_per_tile,), jnp.float32),
            pltpu.VMEM((_LANES,), jnp.float32),
            pltpu.SemaphoreType.DMA,
        ],
    )
    def sc_pool(scores_hbm, ids_hbm, b_hbm, out_hbm,
                idx_v, vals_v, outb_v, b_v, sem):
        wid = lax.axis_index("s") * _NC + lax.axis_index("c")
        pltpu.sync_copy(ids_hbm.at[wid], idx_v)
        pltpu.sync_copy(b_hbm, b_v)

        def fire(j, carry):
            pltpu.async_copy(scores_hbm.at[idx_v.at[j]], vals_v.at[j], sem)
            return carry

        lax.fori_loop(0, S, fire, 0)

        def drain(j, carry):
            pltpu.make_async_copy(scores_hbm.at[idx_v.at[j]],
                                  vals_v.at[j], sem).wait()
            return carry

        lax.fori_loop(0, S, drain, 0)

        bias = b_v[...]
        for g in range(groups):
            def red(s, acc, _g=g):
                return acc + vals_v[s, pl.ds(_g * _LANES, _LANES)]

            acc = lax.fori_loop(0, S, red, jnp.zeros((_LANES,), jnp.float32))
            outb_v[pl.ds(g * _LANES, _LANES)] = acc + bias

        pltpu.sync_copy(outb_v, out_hbm.at[pl.ds(wid * rows_per_tile,
                                                 rows_per_tile)])

    return sc_pool


def kernel(input_ids, embedding, head_w, head_b):
    B, S = input_ids.shape
    V, D = embedding.shape
    assert D == 32 and B % (_NW * _LANES) == 0 and V % _FOLD == 0
    N = V // _FOLD

    emb8 = embedding.reshape(N, _FOLD * D)
    k = jnp.arange(_FOLD * D, dtype=jnp.int32)
    wrep = jnp.where(
        (k[None, :] // D) == jnp.arange(_FOLD, dtype=jnp.int32)[:, None],
        jnp.tile(head_w.astype(jnp.float32).reshape(D) * (1.0 / S),
                 _FOLD)[None, :],
        0.0)
    scores = _scores_tc(emb8, wrep).reshape(V)

    rows_per_tile = B // _NW
    pids = (input_ids % _FOLD) * N + input_ids // _FOLD
    ids_t = jnp.swapaxes(pids.reshape(_NW, rows_per_tile, S), 1, 2)
    b16 = jnp.broadcast_to(head_b.astype(jnp.float32), (_LANES,))

    return _make_sc_pool(B, S)(scores, ids_t, b16)


# speedup vs baseline: 2.1136x; 1.0228x over previous
import functools

import jax
import jax.numpy as jnp
from jax import lax
from jax.experimental import pallas as pl
from jax.experimental.pallas import tpu as pltpu
from jax.experimental.pallas import tpu_sc as plsc

_NC = 2
_NS = 16
_NW = _NC * _NS
_LANES = 16
_FOLD = 8
_BLKR = 8192


def _scores_tc(emb8, wrep):
    N, K = emb8.shape
    grid = (N + _BLKR - 1) // _BLKR

    def body(w_ref, e_ref, o_ref):
        g = lax.dot_general(
            w_ref[...], e_ref[...],
            (((1,), (1,)), ((), ())),
            preferred_element_type=jnp.float32,
            precision=lax.Precision.HIGHEST)
        o_ref[...] = g.reshape(_FOLD, _BLKR // 128, 128).swapaxes(0, 1) \
                      .reshape(_FOLD * _BLKR)

    return pl.pallas_call(
        body,
        grid=(grid,),
        in_specs=[
            pl.BlockSpec((_FOLD, K), lambda i: (0, 0)),
            pl.BlockSpec((_BLKR, K), lambda i: (i, 0)),
        ],
        out_specs=pl.BlockSpec((_FOLD * _BLKR,), lambda i: (i,)),
        out_shape=jax.ShapeDtypeStruct((grid * _FOLD * _BLKR,), jnp.float32),
    )(wrep, emb8)


def _make_sc_pool(B, S):
    rows_per_tile = B // _NW
    groups = rows_per_tile // _LANES
    mesh = plsc.VectorSubcoreMesh(core_axis_name="c", subcore_axis_name="s")

    @functools.partial(
        pl.kernel,
        mesh=mesh,
        out_type=jax.ShapeDtypeStruct((B,), jnp.float32),
        scratch_types=[
            pltpu.VMEM((S, rows_per_tile), jnp.int32),
            pltpu.VMEM((S, rows_per_tile), jnp.float32),
            pltpu.VMEM((rows_per_tile,), jnp.float32),
            pltpu.VMEM((_LANES,), jnp.float32),
            pltpu.SemaphoreType.DMA,
        ],
    )
    def sc_pool(scores_hbm, ids_hbm, b_hbm, out_hbm,
                idx_v, vals_v, outb_v, b_v, sem):
        wid = lax.axis_index("s") * _NC + lax.axis_index("c")
        pltpu.sync_copy(ids_hbm.at[wid], idx_v)
        pltpu.sync_copy(b_hbm, b_v)

        def fire(j, carry):
            pltpu.async_copy(scores_hbm.at[idx_v.at[j]], vals_v.at[j], sem)
            return carry

        lax.fori_loop(0, S, fire, 0)

        def drain(j, carry):
            pltpu.make_async_copy(scores_hbm.at[idx_v.at[j]],
                                  vals_v.at[j], sem).wait()
            return carry

        lax.fori_loop(0, S, drain, 0)

        bias = b_v[...]
        for g in range(groups):
            def red(s, acc, _g=g):
                return acc + vals_v[s, pl.ds(_g * _LANES, _LANES)]

            acc = lax.fori_loop(0, S, red, jnp.zeros((_LANES,), jnp.float32))
            outb_v[pl.ds(g * _LANES, _LANES)] = acc + bias

        pltpu.sync_copy(outb_v, out_hbm.at[pl.ds(wid * rows_per_tile,
                                                 rows_per_tile)])

    return sc_pool


def kernel(input_ids, embedding, head_w, head_b):
    B, S = input_ids.shape
    V, D = embedding.shape
    assert D == 32 and B % (_NW * _LANES) == 0 and V % _FOLD == 0
    N = V // _FOLD

    emb8 = embedding.reshape(N, _FOLD * D)
    k = jnp.arange(_FOLD * D, dtype=jnp.int32)
    wrep = jnp.where(
        (k[None, :] // D) == jnp.arange(_FOLD, dtype=jnp.int32)[:, None],
        jnp.tile(head_w.astype(jnp.float32).reshape(D) * (1.0 / S),
                 _FOLD)[None, :],
        0.0)
    scores = _scores_tc(emb8, wrep)

    rows_per_tile = B // _NW
    r = input_ids // _FOLD
    pids = (r // 128) * 1024 + (input_ids % _FOLD) * 128 + (r % 128)
    ids_t = jnp.swapaxes(pids.reshape(_NW, rows_per_tile, S), 1, 2)
    b16 = jnp.broadcast_to(head_b.astype(jnp.float32), (_LANES,))

    return _make_sc_pool(B, S)(scores, ids_t, b16)
